# baseline (device time: 105877 ns/iter reference)
import jax
import jax.numpy as jnp
from jax import lax
from jax.experimental import pallas as pl
from jax.experimental.pallas import tpu as pltpu

N_DEV = 4


def kernel(x, w_mat, scale_x, scale_w):
    m_per, k = x.shape
    n_per = w_mat.shape[1]
    m_glob = N_DEV * m_per
    half = m_per // 2

    x8_in = x.astype(jnp.float8_e4m3fn)
    w8_in = w_mat.astype(jnp.float8_e4m3fn)
    alpha = (scale_x * scale_w).reshape(1, 1)

    def body(x8, w8, alpha_ref, out_ref,
             bufL, bufR, bufO, send_sems, recv_sems):
        my = lax.axis_index("i")
        left = lax.rem(my + N_DEV - 1, N_DEV)
        right = lax.rem(my + 1, N_DEV)

        barrier_sem = pltpu.get_barrier_semaphore()
        for nbr in (left, right):
            pl.semaphore_signal(
                barrier_sem, inc=1,
                device_id=(nbr,), device_id_type=pl.DeviceIdType.MESH,
            )
        pl.semaphore_wait(barrier_sem, 2)

        a = alpha_ref[0, 0]

        def block_out(origin, src):
            acc = jnp.dot(src, w8[...], preferred_element_type=jnp.float32)
            out_ref[pl.ds(origin * m_per, m_per), :] = acc * a

        r1 = pltpu.make_async_remote_copy(
            src_ref=x8, dst_ref=bufL,
            send_sem=send_sems.at[0], recv_sem=recv_sems.at[0],
            device_id=(right,), device_id_type=pl.DeviceIdType.MESH,
        )
        l1 = pltpu.make_async_remote_copy(
            src_ref=x8, dst_ref=bufR,
            send_sem=send_sems.at[1], recv_sem=recv_sems.at[1],
            device_id=(left,), device_id_type=pl.DeviceIdType.MESH,
        )
        r1.start()
        l1.start()

        block_out(my, x8[...])

        r1.wait_recv()
        r2 = pltpu.make_async_remote_copy(
            src_ref=bufL.at[pl.ds(0, half)],
            dst_ref=bufO.at[pl.ds(0, half)],
            send_sem=send_sems.at[2], recv_sem=recv_sems.at[2],
            device_id=(right,), device_id_type=pl.DeviceIdType.MESH,
        )
        r2.start()
        block_out(left, bufL[...])

        l1.wait_recv()
        l2 = pltpu.make_async_remote_copy(
            src_ref=bufR.at[pl.ds(half, half)],
            dst_ref=bufO.at[pl.ds(half, half)],
            send_sem=send_sems.at[3], recv_sem=recv_sems.at[3],
            device_id=(left,), device_id_type=pl.DeviceIdType.MESH,
        )
        l2.start()
        block_out(right, bufR[...])

        r2.wait_recv()
        l2.wait_recv()
        block_out(lax.rem(my + 2, N_DEV), bufO[...])

        r1.wait_send()
        l1.wait_send()
        r2.wait_send()
        l2.wait_send()

    return pl.pallas_call(
        body,
        out_shape=jax.ShapeDtypeStruct((m_glob, n_per), jnp.float32),
        in_specs=[
            pl.BlockSpec(memory_space=pltpu.VMEM),
            pl.BlockSpec(memory_space=pltpu.VMEM),
            pl.BlockSpec(memory_space=pltpu.VMEM),
        ],
        out_specs=pl.BlockSpec(memory_space=pltpu.VMEM),
        scratch_shapes=[
            pltpu.VMEM((m_per, k), jnp.float8_e4m3fn),
            pltpu.VMEM((m_per, k), jnp.float8_e4m3fn),
            pltpu.VMEM((m_per, k), jnp.float8_e4m3fn),
            pltpu.SemaphoreType.DMA((4,)),
            pltpu.SemaphoreType.DMA((4,)),
        ],
        compiler_params=pltpu.CompilerParams(
            collective_id=0,
            vmem_limit_bytes=100 * 1024 * 1024,
        ),
    )(x8_in, w8_in, alpha)


# device time: 86946 ns/iter; 1.2177x vs baseline; 1.2177x over previous
import jax
import jax.numpy as jnp
from jax import lax
from jax.experimental import pallas as pl
from jax.experimental.pallas import tpu as pltpu

N_DEV = 4
N_Q = 4


def kernel(x, w_mat, scale_x, scale_w):
    m_per, k = x.shape
    n_per = w_mat.shape[1]
    m_glob = N_DEV * m_per
    half = m_per // 2
    mq = m_per // N_Q

    alpha = (scale_x * scale_w).reshape(1, 1)

    def body(x_hbm, w_hbm, alpha_ref, out_ref,
             xf, wf, x8, w8, ob, bufL, bufR, bufO,
             dsem, osem, send_sems, recv_sems):
        my = lax.axis_index("i")
        left = lax.rem(my + N_DEV - 1, N_DEV)
        right = lax.rem(my + 1, N_DEV)

        cpx = [pltpu.make_async_copy(
            x_hbm.at[pl.ds(q * mq, mq)], xf.at[pl.ds(q * mq, mq)],
            dsem.at[q]) for q in range(N_Q)]
        cpw = pltpu.make_async_copy(w_hbm, wf, dsem.at[N_Q])
        for c in cpx:
            c.start()
        cpw.start()

        barrier_sem = pltpu.get_barrier_semaphore()
        for nbr in (left, right):
            pl.semaphore_signal(
                barrier_sem, inc=1,
                device_id=(nbr,), device_id_type=pl.DeviceIdType.MESH,
            )
        pl.semaphore_wait(barrier_sem, 2)

        a = alpha_ref[0, 0]

        def rcopy(src, dst, si, ri, dev):
            return pltpu.make_async_remote_copy(
                src_ref=src, dst_ref=dst,
                send_sem=send_sems.at[si], recv_sem=recv_sems.at[ri],
                device_id=(dev,), device_id_type=pl.DeviceIdType.MESH)

        def block_out(slot, origin, src):
            acc = jnp.dot(src, w8[...], preferred_element_type=jnp.float32)
            ob[slot] = acc * a
            pltpu.make_async_copy(
                ob.at[slot], out_ref.at[pl.ds(origin * m_per, m_per)],
                osem.at[slot]).start()

        hop1 = []
        for q in range(N_Q):
            qs = pl.ds(q * mq, mq)
            cpx[q].wait()
            x8[qs] = xf[qs].astype(jnp.float8_e4m3fn)
            rq = rcopy(x8.at[qs], bufL.at[qs], q, q, right)
            lq = rcopy(x8.at[qs], bufR.at[qs], N_Q + q, N_Q + q, left)
            rq.start()
            lq.start()
            hop1 += [rq, lq]

        cpw.wait()
        w8[...] = wf[...].astype(jnp.float8_e4m3fn)
        block_out(0, my, x8[...])

        top = pl.ds(0, half)
        bot = pl.ds(half, half)

        hop1[0].wait_recv()
        hop1[2].wait_recv()
        r2 = rcopy(bufL.at[top], bufO.at[top], 2 * N_Q, 2 * N_Q, right)
        r2.start()
        hop1[4].wait_recv()
        hop1[6].wait_recv()
        block_out(1, left, bufL[...])

        hop1[5].wait_recv()
        hop1[7].wait_recv()
        l2 = rcopy(bufR.at[bot], bufO.at[bot], 2 * N_Q + 1, 2 * N_Q + 1, left)
        l2.start()
        hop1[1].wait_recv()
        hop1[3].wait_recv()
        block_out(2, right, bufR[...])

        r2.wait_recv()
        l2.wait_recv()
        block_out(3, lax.rem(my + 2, N_DEV), bufO[...])

        for c in hop1 + [r2, l2]:
            c.wait_send()
        for s in range(4):
            pltpu.make_async_copy(ob.at[s], ob.at[s], osem.at[s]).wait()

    hbm = pl.BlockSpec(memory_space=pltpu.MemorySpace.HBM)
    return pl.pallas_call(
        body,
        out_shape=jax.ShapeDtypeStruct((m_glob, n_per), jnp.float32),
        in_specs=[hbm, hbm, pl.BlockSpec(memory_space=pltpu.VMEM)],
        out_specs=hbm,
        scratch_shapes=[
            pltpu.VMEM((m_per, k), jnp.float32),
            pltpu.VMEM((k, n_per), jnp.float32),
            pltpu.VMEM((m_per, k), jnp.float8_e4m3fn),
            pltpu.VMEM((k, n_per), jnp.float8_e4m3fn),
            pltpu.VMEM((4, m_per, n_per), jnp.float32),
            pltpu.VMEM((m_per, k), jnp.float8_e4m3fn),
            pltpu.VMEM((m_per, k), jnp.float8_e4m3fn),
            pltpu.VMEM((m_per, k), jnp.float8_e4m3fn),
            pltpu.SemaphoreType.DMA((N_Q + 1,)),
            pltpu.SemaphoreType.DMA((4,)),
            pltpu.SemaphoreType.DMA((2 * N_Q + 2,)),
            pltpu.SemaphoreType.DMA((2 * N_Q + 2,)),
        ],
        compiler_params=pltpu.CompilerParams(
            collective_id=0,
            vmem_limit_bytes=100 * 1024 * 1024,
        ),
    )(x, w_mat, alpha)
